# baseline (speedup 1.0000x reference)
import functools

import jax
import jax.numpy as jnp
from jax import lax
from jax.experimental import pallas as pl
from jax.experimental.pallas import tpu as pltpu
from jax.experimental.pallas import tpu_sc as plsc

NC = 2
NS = 16
NW = NC * NS
CH = 128
NUM_GRAPHS = 64

CHW = 64
NSLOT = 4
WV = CHW * NSLOT


def _mesh():
    return plsc.VectorSubcoreMesh(core_axis_name="c", subcore_axis_name="s")


def _make_deg_kernel(NP, G):
    rpt = NP // NS

    @functools.partial(
        pl.kernel,
        out_type=jax.ShapeDtypeStruct((NC * NP,), jnp.float32),
        mesh=_mesh(),
        scratch_types=[
            pltpu.VMEM_SHARED((NP,), jnp.float32),
            pltpu.VMEM((G, CH), jnp.int32),
            pltpu.VMEM((G, CH), jnp.float32),
            pltpu.VMEM((rpt,), jnp.float32),
        ],
    )
    def deg_kernel(dst_hbm, ew_hbm, out_hbm, acc, dst_v, ew_v, zb):
        c = lax.axis_index("c")
        s = lax.axis_index("s")
        wid = c * NS + s
        pltpu.sync_copy(dst_hbm.at[pl.ds(wid * G, G)], dst_v)
        pltpu.sync_copy(ew_hbm.at[pl.ds(wid * G, G)], ew_v)

        def zrow(i, _):
            zb[pl.ds(i * 16, 16)] = jnp.zeros((16,), jnp.float32)
            return 0

        lax.fori_loop(0, rpt // 16, zrow, 0)
        pltpu.sync_copy(zb, acc.at[pl.ds(s * rpt, rpt)])
        plsc.subcore_barrier()

        def chunk(g, _):
            pltpu.sync_copy(ew_v.at[g], acc.at[dst_v.at[g]], add=True)
            return 0

        lax.fori_loop(0, G, chunk, 0)
        plsc.subcore_barrier()
        pltpu.sync_copy(acc.at[pl.ds(s * rpt, rpt)],
                        out_hbm.at[pl.ds(c * NP + s * rpt, rpt)])

    return deg_kernel


def _make_norm_kernel(NP, EP, EPT):
    @functools.partial(
        pl.kernel,
        out_type=jax.ShapeDtypeStruct((EP,), jnp.float32),
        mesh=_mesh(),
        scratch_types=[
            pltpu.VMEM((NP,), jnp.float32),
            pltpu.VMEM((EPT,), jnp.int32),
            pltpu.VMEM((EPT,), jnp.int32),
            pltpu.VMEM((EPT,), jnp.float32),
            pltpu.VMEM((EPT,), jnp.float32),
        ],
        compiler_params=pltpu.CompilerParams(needs_layout_passes=False),
    )
    def norm_kernel(dis_hbm, src_hbm, dst_hbm, ew_hbm, out_hbm,
                    dis_v, src_v, dst_v, ew_v, nrm_v):
        c = lax.axis_index("c")
        s = lax.axis_index("s")
        base = (c * NS + s) * EPT
        pltpu.sync_copy(dis_hbm, dis_v)
        pltpu.sync_copy(src_hbm.at[pl.ds(base, EPT)], src_v)
        pltpu.sync_copy(dst_hbm.at[pl.ds(base, EPT)], dst_v)
        pltpu.sync_copy(ew_hbm.at[pl.ds(base, EPT)], ew_v)

        def vec(i, _):
            sl = pl.ds(i * 16, 16)
            dis_s = plsc.load_gather(dis_v, [src_v[sl]])
            dis_d = plsc.load_gather(dis_v, [dst_v[sl]])
            nrm_v[sl] = dis_s * ew_v[sl] * dis_d
            return 0

        lax.fori_loop(0, EPT // 16, vec, 0)
        pltpu.sync_copy(nrm_v, out_hbm.at[pl.ds(base, EPT)])

    return norm_kernel


def _make_agg_kernel(NP, G, H):
    rpt = NP // NS
    NWV = G // NSLOT
    EPT = G * CHW

    @functools.partial(
        pl.kernel,
        out_type=jax.ShapeDtypeStruct((NC, NP, H), jnp.float32),
        mesh=_mesh(),
        scratch_types=(
            [
                pltpu.VMEM_SHARED((NP, H), jnp.float32),
            ]
            + [pltpu.VMEM((CHW, H), jnp.float32) for _ in range(NSLOT)]
            + [
                pltpu.VMEM((NSLOT, CHW), jnp.int32),
                pltpu.VMEM((NSLOT, CHW), jnp.int32),
                pltpu.VMEM((NSLOT, CHW), jnp.int32),
                pltpu.VMEM((NSLOT, CHW), jnp.int32),
                pltpu.VMEM((WV,), jnp.float32),
                pltpu.VMEM((WV,), jnp.float32),
            ]
            + [pltpu.SemaphoreType.DMA for _ in range(NSLOT + 2)]
        ),
    )
    def agg_kernel(m_hbm, src_hbm, dst_hbm, nrm_hbm, out_hbm,
                   acc, *rest):
        r = rest[:NSLOT]
        sbA, sbB, dbA, dbB, nbA, nbB = rest[NSLOT:NSLOT + 6]
        gsem = rest[NSLOT + 6:2 * NSLOT + 6]
        ssemA, ssemB = rest[2 * NSLOT + 6:]
        c = lax.axis_index("c")
        s = lax.axis_index("s")
        wid = c * NS + s
        tb = wid * EPT

        def zrow(i, _):
            for cc in range(H // 16):
                r[0][i, pl.ds(cc * 16, 16)] = jnp.zeros((16,), jnp.float32)
            return 0

        lax.fori_loop(0, CHW, zrow, 0)
        for j in range(rpt // CHW):
            pltpu.sync_copy(r[0], acc.at[pl.ds(s * rpt + j * CHW, CHW)])

        pltpu.sync_copy(nrm_hbm.at[pl.ds(tb, WV)], nbA)
        for k in range(NSLOT):
            pltpu.sync_copy(src_hbm.at[pl.ds(tb + k * CHW, CHW)], sbA.at[k])
            pltpu.sync_copy(dst_hbm.at[pl.ds(tb + k * CHW, CHW)], dbA.at[k])
        plsc.subcore_barrier()

        def wait_bytes(sem, dummy_src, dst):
            pltpu.make_async_copy(dummy_src, dst, sem).wait()

        hdummy = m_hbm.at[pl.ds(0, CHW)]

        def scale_slot(rk, nb, k):
            def grp(j, _):
                nv = nb[pl.ds(k * CHW + j * 16, 16)]
                rbase = j * 16
                for kk in range(16):
                    sc = nv[kk]
                    for cc in range(H // 16):
                        sl = pl.ds(cc * 16, 16)
                        rk[rbase + kk, sl] = rk[rbase + kk, sl] * sc
                return 0

            lax.fori_loop(0, CHW // 16, grp, 0)

        def wave(w, sb, db, nb, ssem, sbn, dbn, nbn, ssemn):
            eb = tb + w * WV
            @pl.when(w >= 1)
            def _():
                wait_bytes(ssem, nrm_hbm.at[pl.ds(0, WV)], nb)
                for k in range(NSLOT):
                    wait_bytes(ssem, src_hbm.at[pl.ds(0, CHW)], sb.at[k])
                    wait_bytes(ssem, dst_hbm.at[pl.ds(0, CHW)], db.at[k])

            for k in range(NSLOT):
                pltpu.async_copy(m_hbm.at[sb.at[k]], r[k], gsem[k])

            @pl.when(w + 1 < NWV)
            def _():
                nxt = eb + WV
                pltpu.async_copy(nrm_hbm.at[pl.ds(nxt, WV)], nbn, ssemn)
                for k in range(NSLOT):
                    pltpu.async_copy(src_hbm.at[pl.ds(nxt + k * CHW, CHW)],
                                     sbn.at[k], ssemn)
                    pltpu.async_copy(dst_hbm.at[pl.ds(nxt + k * CHW, CHW)],
                                     dbn.at[k], ssemn)

            for k in range(NSLOT):
                wait_bytes(gsem[k], hdummy, r[k])
                scale_slot(r[k], nb, k)
                pltpu.sync_copy(r[k], acc.at[db.at[k]], add=True)

        def pair(p, _):
            wave(2 * p, sbA, dbA, nbA, ssemA, sbB, dbB, nbB, ssemB)
            wave(2 * p + 1, sbB, dbB, nbB, ssemB, sbA, dbA, nbA, ssemA)
            return 0

        lax.fori_loop(0, NWV // 2, pair, 0)
        plsc.subcore_barrier()
        pltpu.sync_copy(acc.at[pl.ds(s * rpt, rpt)],
                        out_hbm.at[c, pl.ds(s * rpt, rpt)])

    return agg_kernel


def _tc_first(xp, W1, d0, d1, NP, D, H, BM=256):
    def body(x_ref, w_ref, d0_ref, d1_ref, m_ref, dis_ref):
        m_ref[...] = jnp.dot(x_ref[...], w_ref[...],
                             preferred_element_type=jnp.float32)
        deg = d0_ref[...] + d1_ref[...]
        pos = deg > 0
        dis_ref[...] = jnp.where(pos, lax.rsqrt(jnp.where(pos, deg, 1.0)), 0.0)

    return pl.pallas_call(
        body,
        grid=(NP // BM,),
        in_specs=[
            pl.BlockSpec((BM, D), lambda i: (i, 0)),
            pl.BlockSpec((D, H), lambda i: (0, 0)),
            pl.BlockSpec((1, BM), lambda i: (0, i)),
            pl.BlockSpec((1, BM), lambda i: (0, i)),
        ],
        out_specs=[
            pl.BlockSpec((BM, H), lambda i: (i, 0)),
            pl.BlockSpec((1, BM), lambda i: (0, i)),
        ],
        out_shape=[
            jax.ShapeDtypeStruct((NP, H), jnp.float32),
            jax.ShapeDtypeStruct((1, NP), jnp.float32),
        ],
    )(xp, W1, d0, d1)


def _tc_mid(p, b, W, NP, H, BM=256):
    def body(p_ref, b_ref, w_ref, out_ref):
        pv = p_ref[...]
        h = jnp.maximum(pv[0] + pv[1] + b_ref[...], 0.0)
        out_ref[...] = jnp.dot(h, w_ref[...], preferred_element_type=jnp.float32)

    return pl.pallas_call(
        body,
        grid=(NP // BM,),
        in_specs=[
            pl.BlockSpec((2, BM, H), lambda i: (0, i, 0)),
            pl.BlockSpec((1, H), lambda i: (0, 0)),
            pl.BlockSpec((H, H), lambda i: (0, 0)),
        ],
        out_specs=pl.BlockSpec((BM, H), lambda i: (i, 0)),
        out_shape=jax.ShapeDtypeStruct((NP, H), jnp.float32),
    )(p, b, W)


def _tc_final(p, b3, btp, Wl1, bl1, Wl2, bl2, NP, H, BM=256):
    L = Wl1.shape[1]
    C = Wl2.shape[1]
    B = NUM_GRAPHS
    nsteps = NP // BM

    def body(p_ref, b_ref, bt_ref, wl1_ref, bl1_ref, wl2_ref, bl2_ref,
             out_ref, gacc):
        i = pl.program_id(0)

        @pl.when(i == 0)
        def _():
            gacc[...] = jnp.full((B, H), -jnp.inf, jnp.float32)

        pv = p_ref[...]
        h = jnp.maximum(pv[0] + pv[1] + b_ref[...], 0.0)
        m = bt_ref[...]
        for seg in range(B):
            row = jnp.max(jnp.where(m == seg, h, -jnp.inf), axis=0,
                          keepdims=True)
            gacc[pl.ds(seg, 1), :] = jnp.maximum(gacc[pl.ds(seg, 1), :], row)

        @pl.when(i == nsteps - 1)
        def _():
            g = gacc[...]
            z = jnp.maximum(
                jnp.dot(g, wl1_ref[...], preferred_element_type=jnp.float32)
                + bl1_ref[...], 0.0)
            out_ref[...] = (
                jnp.dot(z, wl2_ref[...], preferred_element_type=jnp.float32)
                + bl2_ref[...])

    return pl.pallas_call(
        body,
        grid=(nsteps,),
        in_specs=[
            pl.BlockSpec((2, BM, H), lambda i: (0, i, 0)),
            pl.BlockSpec((1, H), lambda i: (0, 0)),
            pl.BlockSpec((BM, 1), lambda i: (i, 0)),
            pl.BlockSpec((H, L), lambda i: (0, 0)),
            pl.BlockSpec((1, L), lambda i: (0, 0)),
            pl.BlockSpec((L, C), lambda i: (0, 0)),
            pl.BlockSpec((1, C), lambda i: (0, 0)),
        ],
        out_specs=pl.BlockSpec((B, C), lambda i: (0, 0)),
        out_shape=jax.ShapeDtypeStruct((B, C), jnp.float32),
        scratch_shapes=[pltpu.VMEM((B, H), jnp.float32)],
    )(p, b3, btp, Wl1, bl1, Wl2, bl2)


def kernel(x, edge_index, edge_weight, batch, W1, b1, W2, b2, W3, b3,
           Wl1, bl1, Wl2, bl2):
    N, D = x.shape
    E = edge_index.shape[1]
    H = W1.shape[1]

    NP = ((N + 2047) // 2048) * 2048
    EF = E + N
    EP = ((EF + 32767) // 32768) * 32768
    NE_ROWS = EP // CH
    EPT_N = EP // NW
    quantum = NW * CHW * NSLOT * 2
    EP_A = ((EF + quantum - 1) // quantum) * quantum
    G = EP_A // (NW * CHW)

    loop = jnp.arange(N, dtype=jnp.int32)
    padi = jnp.zeros((EP - EF,), jnp.int32)
    padf = jnp.zeros((EP - EF,), jnp.float32)
    srcf = jnp.concatenate([edge_index[0], loop, padi])
    dstf = jnp.concatenate([edge_index[1], loop, padi])
    ewf = jnp.concatenate([edge_weight.reshape(-1).astype(jnp.float32),
                           jnp.ones((N,), jnp.float32), padf])

    xp = jnp.zeros((NP, D), jnp.float32).at[:N].set(x.astype(jnp.float32))
    btp = jnp.full((NP, 1), NUM_GRAPHS, jnp.int32).at[:N, 0].set(batch)

    deg_k = _make_deg_kernel(NP, NE_ROWS // NW)
    norm_k = _make_norm_kernel(NP, EP, EPT_N)
    agg_k = _make_agg_kernel(NP, G, H)

    degp = deg_k(dstf.reshape(NE_ROWS, CH), ewf.reshape(NE_ROWS, CH))
    Msp, dis = _tc_first(xp, W1, degp[:NP].reshape(1, NP),
                         degp[NP:].reshape(1, NP), NP, D, H)
    nrm = norm_k(dis.reshape(NP), srcf, dstf, ewf)

    P = agg_k(Msp, srcf, dstf, nrm)
    Msp = _tc_mid(P, b1.reshape(1, H), W2, NP, H)
    P = agg_k(Msp, srcf, dstf, nrm)
    Msp = _tc_mid(P, b2.reshape(1, H), W3, NP, H)
    P = agg_k(Msp, srcf, dstf, nrm)

    return _tc_final(P, b3.reshape(1, H), btp,
                     Wl1, bl1.reshape(1, Wl1.shape[1]),
                     Wl2, bl2.reshape(1, Wl2.shape[1]), NP, H)


# speedup vs baseline: 1.0352x; 1.0352x over previous
import functools

import jax
import jax.numpy as jnp
from jax import lax
from jax.experimental import pallas as pl
from jax.experimental.pallas import tpu as pltpu
from jax.experimental.pallas import tpu_sc as plsc

NC = 2
NS = 16
NW = NC * NS
CH = 128
NUM_GRAPHS = 64

CHW = 16
NSLOT = 16
WV = CHW * NSLOT


def _mesh():
    return plsc.VectorSubcoreMesh(core_axis_name="c", subcore_axis_name="s")


def _make_deg_kernel(NP, G):
    rpt = NP // NS

    @functools.partial(
        pl.kernel,
        out_type=jax.ShapeDtypeStruct((NC * NP,), jnp.float32),
        mesh=_mesh(),
        scratch_types=[
            pltpu.VMEM_SHARED((NP,), jnp.float32),
            pltpu.VMEM((G, CH), jnp.int32),
            pltpu.VMEM((G, CH), jnp.float32),
            pltpu.VMEM((rpt,), jnp.float32),
        ],
    )
    def deg_kernel(dst_hbm, ew_hbm, out_hbm, acc, dst_v, ew_v, zb):
        c = lax.axis_index("c")
        s = lax.axis_index("s")
        wid = c * NS + s
        pltpu.sync_copy(dst_hbm.at[pl.ds(wid * G, G)], dst_v)
        pltpu.sync_copy(ew_hbm.at[pl.ds(wid * G, G)], ew_v)

        def zrow(i, _):
            zb[pl.ds(i * 16, 16)] = jnp.zeros((16,), jnp.float32)
            return 0

        lax.fori_loop(0, rpt // 16, zrow, 0)
        pltpu.sync_copy(zb, acc.at[pl.ds(s * rpt, rpt)])
        plsc.subcore_barrier()

        def chunk(g, _):
            pltpu.sync_copy(ew_v.at[g], acc.at[dst_v.at[g]], add=True)
            return 0

        lax.fori_loop(0, G, chunk, 0)
        plsc.subcore_barrier()
        pltpu.sync_copy(acc.at[pl.ds(s * rpt, rpt)],
                        out_hbm.at[pl.ds(c * NP + s * rpt, rpt)])

    return deg_kernel


def _make_norm_kernel(NP, EP, EPT):
    @functools.partial(
        pl.kernel,
        out_type=jax.ShapeDtypeStruct((EP,), jnp.float32),
        mesh=_mesh(),
        scratch_types=[
            pltpu.VMEM((NP,), jnp.float32),
            pltpu.VMEM((EPT,), jnp.int32),
            pltpu.VMEM((EPT,), jnp.int32),
            pltpu.VMEM((EPT,), jnp.float32),
            pltpu.VMEM((EPT,), jnp.float32),
        ],
        compiler_params=pltpu.CompilerParams(needs_layout_passes=False),
    )
    def norm_kernel(dis_hbm, src_hbm, dst_hbm, ew_hbm, out_hbm,
                    dis_v, src_v, dst_v, ew_v, nrm_v):
        c = lax.axis_index("c")
        s = lax.axis_index("s")
        base = (c * NS + s) * EPT
        pltpu.sync_copy(dis_hbm, dis_v)
        pltpu.sync_copy(src_hbm.at[pl.ds(base, EPT)], src_v)
        pltpu.sync_copy(dst_hbm.at[pl.ds(base, EPT)], dst_v)
        pltpu.sync_copy(ew_hbm.at[pl.ds(base, EPT)], ew_v)

        def vec(i, _):
            sl = pl.ds(i * 16, 16)
            dis_s = plsc.load_gather(dis_v, [src_v[sl]])
            dis_d = plsc.load_gather(dis_v, [dst_v[sl]])
            nrm_v[sl] = dis_s * ew_v[sl] * dis_d
            return 0

        lax.fori_loop(0, EPT // 16, vec, 0)
        pltpu.sync_copy(nrm_v, out_hbm.at[pl.ds(base, EPT)])

    return norm_kernel


def _make_agg_kernel(NP, G, H):
    rpt = NP // NS
    NWV = G // NSLOT
    EPT = G * CHW

    @functools.partial(
        pl.kernel,
        out_type=jax.ShapeDtypeStruct((NC, NP, H), jnp.float32),
        mesh=_mesh(),
        scratch_types=(
            [
                pltpu.VMEM_SHARED((NP, H), jnp.float32),
            ]
            + [pltpu.VMEM((CHW, H), jnp.float32) for _ in range(NSLOT)]
            + [
                pltpu.VMEM((NSLOT, CHW), jnp.int32),
                pltpu.VMEM((NSLOT, CHW), jnp.int32),
                pltpu.VMEM((NSLOT, CHW), jnp.int32),
                pltpu.VMEM((NSLOT, CHW), jnp.int32),
                pltpu.VMEM((WV,), jnp.float32),
                pltpu.VMEM((WV,), jnp.float32),
            ]
            + [pltpu.SemaphoreType.DMA for _ in range(NSLOT + 2)]
        ),
    )
    def agg_kernel(m_hbm, src_hbm, dst_hbm, nrm_hbm, out_hbm,
                   acc, *rest):
        r = rest[:NSLOT]
        sbA, sbB, dbA, dbB, nbA, nbB = rest[NSLOT:NSLOT + 6]
        gsem = rest[NSLOT + 6:2 * NSLOT + 6]
        ssemA, ssemB = rest[2 * NSLOT + 6:]
        c = lax.axis_index("c")
        s = lax.axis_index("s")
        wid = c * NS + s
        tb = wid * EPT

        def zrow(i, _):
            for cc in range(H // 16):
                r[0][i, pl.ds(cc * 16, 16)] = jnp.zeros((16,), jnp.float32)
            return 0

        lax.fori_loop(0, CHW, zrow, 0)
        for j in range(rpt // CHW):
            pltpu.sync_copy(r[0], acc.at[pl.ds(s * rpt + j * CHW, CHW)])

        pltpu.sync_copy(nrm_hbm.at[pl.ds(tb, WV)], nbA)
        for k in range(NSLOT):
            pltpu.sync_copy(src_hbm.at[pl.ds(tb + k * CHW, CHW)], sbA.at[k])
            pltpu.sync_copy(dst_hbm.at[pl.ds(tb + k * CHW, CHW)], dbA.at[k])
        plsc.subcore_barrier()

        def wait_bytes(sem, dummy_src, dst):
            pltpu.make_async_copy(dummy_src, dst, sem).wait()

        hdummy = m_hbm.at[pl.ds(0, CHW)]

        def scale_slot(rk, nb, k):
            def grp(j, _):
                nv = nb[pl.ds(k * CHW + j * 16, 16)]
                rbase = j * 16
                for kk in range(16):
                    sc = nv[kk]
                    for cc in range(H // 16):
                        sl = pl.ds(cc * 16, 16)
                        rk[rbase + kk, sl] = rk[rbase + kk, sl] * sc
                return 0

            lax.fori_loop(0, CHW // 16, grp, 0)

        def wave(w, sb, db, nb, ssem, sbn, dbn, nbn, ssemn):
            eb = tb + w * WV
            @pl.when(w >= 1)
            def _():
                wait_bytes(ssem, nrm_hbm.at[pl.ds(0, WV)], nb)
                for k in range(NSLOT):
                    wait_bytes(ssem, src_hbm.at[pl.ds(0, CHW)], sb.at[k])
                    wait_bytes(ssem, dst_hbm.at[pl.ds(0, CHW)], db.at[k])

            for k in range(NSLOT):
                pltpu.async_copy(m_hbm.at[sb.at[k]], r[k], gsem[k])

            @pl.when(w + 1 < NWV)
            def _():
                nxt = eb + WV
                pltpu.async_copy(nrm_hbm.at[pl.ds(nxt, WV)], nbn, ssemn)
                for k in range(NSLOT):
                    pltpu.async_copy(src_hbm.at[pl.ds(nxt + k * CHW, CHW)],
                                     sbn.at[k], ssemn)
                    pltpu.async_copy(dst_hbm.at[pl.ds(nxt + k * CHW, CHW)],
                                     dbn.at[k], ssemn)

            for k in range(NSLOT):
                wait_bytes(gsem[k], hdummy, r[k])
                scale_slot(r[k], nb, k)
                pltpu.sync_copy(r[k], acc.at[db.at[k]], add=True)

        def pair(p, _):
            wave(2 * p, sbA, dbA, nbA, ssemA, sbB, dbB, nbB, ssemB)
            wave(2 * p + 1, sbB, dbB, nbB, ssemB, sbA, dbA, nbA, ssemA)
            return 0

        lax.fori_loop(0, NWV // 2, pair, 0)
        plsc.subcore_barrier()
        pltpu.sync_copy(acc.at[pl.ds(s * rpt, rpt)],
                        out_hbm.at[c, pl.ds(s * rpt, rpt)])

    return agg_kernel


def _tc_first(xp, W1, d0, d1, NP, D, H, BM=256):
    def body(x_ref, w_ref, d0_ref, d1_ref, m_ref, dis_ref):
        m_ref[...] = jnp.dot(x_ref[...], w_ref[...],
                             preferred_element_type=jnp.float32)
        deg = d0_ref[...] + d1_ref[...]
        pos = deg > 0
        dis_ref[...] = jnp.where(pos, lax.rsqrt(jnp.where(pos, deg, 1.0)), 0.0)

    return pl.pallas_call(
        body,
        grid=(NP // BM,),
        in_specs=[
            pl.BlockSpec((BM, D), lambda i: (i, 0)),
            pl.BlockSpec((D, H), lambda i: (0, 0)),
            pl.BlockSpec((1, BM), lambda i: (0, i)),
            pl.BlockSpec((1, BM), lambda i: (0, i)),
        ],
        out_specs=[
            pl.BlockSpec((BM, H), lambda i: (i, 0)),
            pl.BlockSpec((1, BM), lambda i: (0, i)),
        ],
        out_shape=[
            jax.ShapeDtypeStruct((NP, H), jnp.float32),
            jax.ShapeDtypeStruct((1, NP), jnp.float32),
        ],
    )(xp, W1, d0, d1)


def _tc_mid(p, b, W, NP, H, BM=256):
    def body(p_ref, b_ref, w_ref, out_ref):
        pv = p_ref[...]
        h = jnp.maximum(pv[0] + pv[1] + b_ref[...], 0.0)
        out_ref[...] = jnp.dot(h, w_ref[...], preferred_element_type=jnp.float32)

    return pl.pallas_call(
        body,
        grid=(NP // BM,),
        in_specs=[
            pl.BlockSpec((2, BM, H), lambda i: (0, i, 0)),
            pl.BlockSpec((1, H), lambda i: (0, 0)),
            pl.BlockSpec((H, H), lambda i: (0, 0)),
        ],
        out_specs=pl.BlockSpec((BM, H), lambda i: (i, 0)),
        out_shape=jax.ShapeDtypeStruct((NP, H), jnp.float32),
    )(p, b, W)


def _tc_final(p, b3, btp, Wl1, bl1, Wl2, bl2, NP, H, BM=256):
    L = Wl1.shape[1]
    C = Wl2.shape[1]
    B = NUM_GRAPHS
    nsteps = NP // BM

    def body(p_ref, b_ref, bt_ref, wl1_ref, bl1_ref, wl2_ref, bl2_ref,
             out_ref, gacc):
        i = pl.program_id(0)

        @pl.when(i == 0)
        def _():
            gacc[...] = jnp.full((B, H), -jnp.inf, jnp.float32)

        pv = p_ref[...]
        h = jnp.maximum(pv[0] + pv[1] + b_ref[...], 0.0)
        m = bt_ref[...]
        for seg in range(B):
            row = jnp.max(jnp.where(m == seg, h, -jnp.inf), axis=0,
                          keepdims=True)
            gacc[pl.ds(seg, 1), :] = jnp.maximum(gacc[pl.ds(seg, 1), :], row)

        @pl.when(i == nsteps - 1)
        def _():
            g = gacc[...]
            z = jnp.maximum(
                jnp.dot(g, wl1_ref[...], preferred_element_type=jnp.float32)
                + bl1_ref[...], 0.0)
            out_ref[...] = (
                jnp.dot(z, wl2_ref[...], preferred_element_type=jnp.float32)
                + bl2_ref[...])

    return pl.pallas_call(
        body,
        grid=(nsteps,),
        in_specs=[
            pl.BlockSpec((2, BM, H), lambda i: (0, i, 0)),
            pl.BlockSpec((1, H), lambda i: (0, 0)),
            pl.BlockSpec((BM, 1), lambda i: (i, 0)),
            pl.BlockSpec((H, L), lambda i: (0, 0)),
            pl.BlockSpec((1, L), lambda i: (0, 0)),
            pl.BlockSpec((L, C), lambda i: (0, 0)),
            pl.BlockSpec((1, C), lambda i: (0, 0)),
        ],
        out_specs=pl.BlockSpec((B, C), lambda i: (0, 0)),
        out_shape=jax.ShapeDtypeStruct((B, C), jnp.float32),
        scratch_shapes=[pltpu.VMEM((B, H), jnp.float32)],
    )(p, b3, btp, Wl1, bl1, Wl2, bl2)


def kernel(x, edge_index, edge_weight, batch, W1, b1, W2, b2, W3, b3,
           Wl1, bl1, Wl2, bl2):
    N, D = x.shape
    E = edge_index.shape[1]
    H = W1.shape[1]

    NP = ((N + 2047) // 2048) * 2048
    EF = E + N
    EP = ((EF + 32767) // 32768) * 32768
    NE_ROWS = EP // CH
    EPT_N = EP // NW
    quantum = NW * CHW * NSLOT * 2
    EP_A = ((EF + quantum - 1) // quantum) * quantum
    G = EP_A // (NW * CHW)

    loop = jnp.arange(N, dtype=jnp.int32)
    padi = jnp.zeros((EP - EF,), jnp.int32)
    padf = jnp.zeros((EP - EF,), jnp.float32)
    srcf = jnp.concatenate([edge_index[0], loop, padi])
    dstf = jnp.concatenate([edge_index[1], loop, padi])
    ewf = jnp.concatenate([edge_weight.reshape(-1).astype(jnp.float32),
                           jnp.ones((N,), jnp.float32), padf])

    xp = jnp.zeros((NP, D), jnp.float32).at[:N].set(x.astype(jnp.float32))
    btp = jnp.full((NP, 1), NUM_GRAPHS, jnp.int32).at[:N, 0].set(batch)

    deg_k = _make_deg_kernel(NP, NE_ROWS // NW)
    norm_k = _make_norm_kernel(NP, EP, EPT_N)
    agg_k = _make_agg_kernel(NP, G, H)

    degp = deg_k(dstf.reshape(NE_ROWS, CH), ewf.reshape(NE_ROWS, CH))
    Msp, dis = _tc_first(xp, W1, degp[:NP].reshape(1, NP),
                         degp[NP:].reshape(1, NP), NP, D, H)
    nrm = norm_k(dis.reshape(NP), srcf, dstf, ewf)

    P = agg_k(Msp, srcf, dstf, nrm)
    Msp = _tc_mid(P, b1.reshape(1, H), W2, NP, H)
    P = agg_k(Msp, srcf, dstf, nrm)
    Msp = _tc_mid(P, b2.reshape(1, H), W3, NP, H)
    P = agg_k(Msp, srcf, dstf, nrm)

    return _tc_final(P, b3.reshape(1, H), btp,
                     Wl1, bl1.reshape(1, Wl1.shape[1]),
                     Wl2, bl2.reshape(1, Wl2.shape[1]), NP, H)


# speedup vs baseline: 1.0373x; 1.0020x over previous
import functools

import jax
import jax.numpy as jnp
from jax import lax
from jax.experimental import pallas as pl
from jax.experimental.pallas import tpu as pltpu
from jax.experimental.pallas import tpu_sc as plsc

NC = 2
NS = 16
NW = NC * NS
CH = 128
NUM_GRAPHS = 64

CHW = 32
NSLOT = 8
WV = CHW * NSLOT


def _mesh():
    return plsc.VectorSubcoreMesh(core_axis_name="c", subcore_axis_name="s")


def _make_deg_kernel(NP, G):
    rpt = NP // NS

    @functools.partial(
        pl.kernel,
        out_type=jax.ShapeDtypeStruct((NC * NP,), jnp.float32),
        mesh=_mesh(),
        scratch_types=[
            pltpu.VMEM_SHARED((NP,), jnp.float32),
            pltpu.VMEM((G, CH), jnp.int32),
            pltpu.VMEM((G, CH), jnp.float32),
            pltpu.VMEM((rpt,), jnp.float32),
        ],
    )
    def deg_kernel(dst_hbm, ew_hbm, out_hbm, acc, dst_v, ew_v, zb):
        c = lax.axis_index("c")
        s = lax.axis_index("s")
        wid = c * NS + s
        pltpu.sync_copy(dst_hbm.at[pl.ds(wid * G, G)], dst_v)
        pltpu.sync_copy(ew_hbm.at[pl.ds(wid * G, G)], ew_v)

        def zrow(i, _):
            zb[pl.ds(i * 16, 16)] = jnp.zeros((16,), jnp.float32)
            return 0

        lax.fori_loop(0, rpt // 16, zrow, 0)
        pltpu.sync_copy(zb, acc.at[pl.ds(s * rpt, rpt)])
        plsc.subcore_barrier()

        def chunk(g, _):
            pltpu.sync_copy(ew_v.at[g], acc.at[dst_v.at[g]], add=True)
            return 0

        lax.fori_loop(0, G, chunk, 0)
        plsc.subcore_barrier()
        pltpu.sync_copy(acc.at[pl.ds(s * rpt, rpt)],
                        out_hbm.at[pl.ds(c * NP + s * rpt, rpt)])

    return deg_kernel


def _make_norm_kernel(NP, EP, EPT):
    @functools.partial(
        pl.kernel,
        out_type=jax.ShapeDtypeStruct((EP,), jnp.float32),
        mesh=_mesh(),
        scratch_types=[
            pltpu.VMEM((NP,), jnp.float32),
            pltpu.VMEM((EPT,), jnp.int32),
            pltpu.VMEM((EPT,), jnp.int32),
            pltpu.VMEM((EPT,), jnp.float32),
            pltpu.VMEM((EPT,), jnp.float32),
        ],
        compiler_params=pltpu.CompilerParams(needs_layout_passes=False),
    )
    def norm_kernel(dis_hbm, src_hbm, dst_hbm, ew_hbm, out_hbm,
                    dis_v, src_v, dst_v, ew_v, nrm_v):
        c = lax.axis_index("c")
        s = lax.axis_index("s")
        base = (c * NS + s) * EPT
        pltpu.sync_copy(dis_hbm, dis_v)
        pltpu.sync_copy(src_hbm.at[pl.ds(base, EPT)], src_v)
        pltpu.sync_copy(dst_hbm.at[pl.ds(base, EPT)], dst_v)
        pltpu.sync_copy(ew_hbm.at[pl.ds(base, EPT)], ew_v)

        def vec(i, _):
            sl = pl.ds(i * 16, 16)
            dis_s = plsc.load_gather(dis_v, [src_v[sl]])
            dis_d = plsc.load_gather(dis_v, [dst_v[sl]])
            nrm_v[sl] = dis_s * ew_v[sl] * dis_d
            return 0

        lax.fori_loop(0, EPT // 16, vec, 0)
        pltpu.sync_copy(nrm_v, out_hbm.at[pl.ds(base, EPT)])

    return norm_kernel


def _make_agg_kernel(NP, G, H):
    rpt = NP // NS
    NWV = G // NSLOT
    EPT = G * CHW

    @functools.partial(
        pl.kernel,
        out_type=jax.ShapeDtypeStruct((NC, NP, H), jnp.float32),
        mesh=_mesh(),
        scratch_types=(
            [
                pltpu.VMEM_SHARED((NP, H), jnp.float32),
            ]
            + [pltpu.VMEM((CHW, H), jnp.float32) for _ in range(NSLOT)]
            + [
                pltpu.VMEM((NSLOT, CHW), jnp.int32),
                pltpu.VMEM((NSLOT, CHW), jnp.int32),
                pltpu.VMEM((NSLOT, CHW), jnp.int32),
                pltpu.VMEM((NSLOT, CHW), jnp.int32),
                pltpu.VMEM((WV,), jnp.float32),
                pltpu.VMEM((WV,), jnp.float32),
            ]
            + [pltpu.SemaphoreType.DMA for _ in range(NSLOT + 2)]
        ),
    )
    def agg_kernel(m_hbm, src_hbm, dst_hbm, nrm_hbm, out_hbm,
                   acc, *rest):
        r = rest[:NSLOT]
        sbA, sbB, dbA, dbB, nbA, nbB = rest[NSLOT:NSLOT + 6]
        gsem = rest[NSLOT + 6:2 * NSLOT + 6]
        ssemA, ssemB = rest[2 * NSLOT + 6:]
        c = lax.axis_index("c")
        s = lax.axis_index("s")
        wid = c * NS + s
        tb = wid * EPT

        def zrow(i, _):
            for cc in range(H // 16):
                r[0][i, pl.ds(cc * 16, 16)] = jnp.zeros((16,), jnp.float32)
            return 0

        lax.fori_loop(0, CHW, zrow, 0)
        for j in range(rpt // CHW):
            pltpu.sync_copy(r[0], acc.at[pl.ds(s * rpt + j * CHW, CHW)])

        pltpu.sync_copy(nrm_hbm.at[pl.ds(tb, WV)], nbA)
        for k in range(NSLOT):
            pltpu.sync_copy(src_hbm.at[pl.ds(tb + k * CHW, CHW)], sbA.at[k])
            pltpu.sync_copy(dst_hbm.at[pl.ds(tb + k * CHW, CHW)], dbA.at[k])
        plsc.subcore_barrier()

        def wait_bytes(sem, dummy_src, dst):
            pltpu.make_async_copy(dummy_src, dst, sem).wait()

        hdummy = m_hbm.at[pl.ds(0, CHW)]

        def scale_slot(rk, nb, k):
            def grp(j, _):
                nv = nb[pl.ds(k * CHW + j * 16, 16)]
                rbase = j * 16
                for kk in range(16):
                    sc = nv[kk]
                    for cc in range(H // 16):
                        sl = pl.ds(cc * 16, 16)
                        rk[rbase + kk, sl] = rk[rbase + kk, sl] * sc
                return 0

            lax.fori_loop(0, CHW // 16, grp, 0)

        def wave(w, sb, db, nb, ssem, sbn, dbn, nbn, ssemn):
            eb = tb + w * WV
            @pl.when(w >= 1)
            def _():
                wait_bytes(ssem, nrm_hbm.at[pl.ds(0, WV)], nb)
                for k in range(NSLOT):
                    wait_bytes(ssem, src_hbm.at[pl.ds(0, CHW)], sb.at[k])
                    wait_bytes(ssem, dst_hbm.at[pl.ds(0, CHW)], db.at[k])

            for k in range(NSLOT):
                pltpu.async_copy(m_hbm.at[sb.at[k]], r[k], gsem[k])

            @pl.when(w + 1 < NWV)
            def _():
                nxt = eb + WV
                pltpu.async_copy(nrm_hbm.at[pl.ds(nxt, WV)], nbn, ssemn)
                for k in range(NSLOT):
                    pltpu.async_copy(src_hbm.at[pl.ds(nxt + k * CHW, CHW)],
                                     sbn.at[k], ssemn)
                    pltpu.async_copy(dst_hbm.at[pl.ds(nxt + k * CHW, CHW)],
                                     dbn.at[k], ssemn)

            for k in range(NSLOT):
                wait_bytes(gsem[k], hdummy, r[k])
                scale_slot(r[k], nb, k)
                pltpu.sync_copy(r[k], acc.at[db.at[k]], add=True)

        def pair(p, _):
            wave(2 * p, sbA, dbA, nbA, ssemA, sbB, dbB, nbB, ssemB)
            wave(2 * p + 1, sbB, dbB, nbB, ssemB, sbA, dbA, nbA, ssemA)
            return 0

        lax.fori_loop(0, NWV // 2, pair, 0)
        plsc.subcore_barrier()
        pltpu.sync_copy(acc.at[pl.ds(s * rpt, rpt)],
                        out_hbm.at[c, pl.ds(s * rpt, rpt)])

    return agg_kernel


def _tc_first(xp, W1, d0, d1, NP, D, H, BM=256):
    def body(x_ref, w_ref, d0_ref, d1_ref, m_ref, dis_ref):
        m_ref[...] = jnp.dot(x_ref[...], w_ref[...],
                             preferred_element_type=jnp.float32)
        deg = d0_ref[...] + d1_ref[...]
        pos = deg > 0
        dis_ref[...] = jnp.where(pos, lax.rsqrt(jnp.where(pos, deg, 1.0)), 0.0)

    return pl.pallas_call(
        body,
        grid=(NP // BM,),
        in_specs=[
            pl.BlockSpec((BM, D), lambda i: (i, 0)),
            pl.BlockSpec((D, H), lambda i: (0, 0)),
            pl.BlockSpec((1, BM), lambda i: (0, i)),
            pl.BlockSpec((1, BM), lambda i: (0, i)),
        ],
        out_specs=[
            pl.BlockSpec((BM, H), lambda i: (i, 0)),
            pl.BlockSpec((1, BM), lambda i: (0, i)),
        ],
        out_shape=[
            jax.ShapeDtypeStruct((NP, H), jnp.float32),
            jax.ShapeDtypeStruct((1, NP), jnp.float32),
        ],
    )(xp, W1, d0, d1)


def _tc_mid(p, b, W, NP, H, BM=256):
    def body(p_ref, b_ref, w_ref, out_ref):
        pv = p_ref[...]
        h = jnp.maximum(pv[0] + pv[1] + b_ref[...], 0.0)
        out_ref[...] = jnp.dot(h, w_ref[...], preferred_element_type=jnp.float32)

    return pl.pallas_call(
        body,
        grid=(NP // BM,),
        in_specs=[
            pl.BlockSpec((2, BM, H), lambda i: (0, i, 0)),
            pl.BlockSpec((1, H), lambda i: (0, 0)),
            pl.BlockSpec((H, H), lambda i: (0, 0)),
        ],
        out_specs=pl.BlockSpec((BM, H), lambda i: (i, 0)),
        out_shape=jax.ShapeDtypeStruct((NP, H), jnp.float32),
    )(p, b, W)


def _tc_final(p, b3, btp, Wl1, bl1, Wl2, bl2, NP, H, BM=256):
    L = Wl1.shape[1]
    C = Wl2.shape[1]
    B = NUM_GRAPHS
    nsteps = NP // BM

    def body(p_ref, b_ref, bt_ref, wl1_ref, bl1_ref, wl2_ref, bl2_ref,
             out_ref, gacc):
        i = pl.program_id(0)

        @pl.when(i == 0)
        def _():
            gacc[...] = jnp.full((B, H), -jnp.inf, jnp.float32)

        pv = p_ref[...]
        h = jnp.maximum(pv[0] + pv[1] + b_ref[...], 0.0)
        m = bt_ref[...]
        for seg in range(B):
            row = jnp.max(jnp.where(m == seg, h, -jnp.inf), axis=0,
                          keepdims=True)
            gacc[pl.ds(seg, 1), :] = jnp.maximum(gacc[pl.ds(seg, 1), :], row)

        @pl.when(i == nsteps - 1)
        def _():
            g = gacc[...]
            z = jnp.maximum(
                jnp.dot(g, wl1_ref[...], preferred_element_type=jnp.float32)
                + bl1_ref[...], 0.0)
            out_ref[...] = (
                jnp.dot(z, wl2_ref[...], preferred_element_type=jnp.float32)
                + bl2_ref[...])

    return pl.pallas_call(
        body,
        grid=(nsteps,),
        in_specs=[
            pl.BlockSpec((2, BM, H), lambda i: (0, i, 0)),
            pl.BlockSpec((1, H), lambda i: (0, 0)),
            pl.BlockSpec((BM, 1), lambda i: (i, 0)),
            pl.BlockSpec((H, L), lambda i: (0, 0)),
            pl.BlockSpec((1, L), lambda i: (0, 0)),
            pl.BlockSpec((L, C), lambda i: (0, 0)),
            pl.BlockSpec((1, C), lambda i: (0, 0)),
        ],
        out_specs=pl.BlockSpec((B, C), lambda i: (0, 0)),
        out_shape=jax.ShapeDtypeStruct((B, C), jnp.float32),
        scratch_shapes=[pltpu.VMEM((B, H), jnp.float32)],
    )(p, b3, btp, Wl1, bl1, Wl2, bl2)


def kernel(x, edge_index, edge_weight, batch, W1, b1, W2, b2, W3, b3,
           Wl1, bl1, Wl2, bl2):
    N, D = x.shape
    E = edge_index.shape[1]
    H = W1.shape[1]

    NP = ((N + 2047) // 2048) * 2048
    EF = E + N
    EP = ((EF + 32767) // 32768) * 32768
    NE_ROWS = EP // CH
    EPT_N = EP // NW
    quantum = NW * CHW * NSLOT * 2
    EP_A = ((EF + quantum - 1) // quantum) * quantum
    G = EP_A // (NW * CHW)

    loop = jnp.arange(N, dtype=jnp.int32)
    padi = jnp.zeros((EP - EF,), jnp.int32)
    padf = jnp.zeros((EP - EF,), jnp.float32)
    srcf = jnp.concatenate([edge_index[0], loop, padi])
    dstf = jnp.concatenate([edge_index[1], loop, padi])
    ewf = jnp.concatenate([edge_weight.reshape(-1).astype(jnp.float32),
                           jnp.ones((N,), jnp.float32), padf])

    xp = jnp.zeros((NP, D), jnp.float32).at[:N].set(x.astype(jnp.float32))
    btp = jnp.full((NP, 1), NUM_GRAPHS, jnp.int32).at[:N, 0].set(batch)

    deg_k = _make_deg_kernel(NP, NE_ROWS // NW)
    norm_k = _make_norm_kernel(NP, EP, EPT_N)
    agg_k = _make_agg_kernel(NP, G, H)

    degp = deg_k(dstf.reshape(NE_ROWS, CH), ewf.reshape(NE_ROWS, CH))
    Msp, dis = _tc_first(xp, W1, degp[:NP].reshape(1, NP),
                         degp[NP:].reshape(1, NP), NP, D, H)
    nrm = norm_k(dis.reshape(NP), srcf, dstf, ewf)

    P = agg_k(Msp, srcf, dstf, nrm)
    Msp = _tc_mid(P, b1.reshape(1, H), W2, NP, H)
    P = agg_k(Msp, srcf, dstf, nrm)
    Msp = _tc_mid(P, b2.reshape(1, H), W3, NP, H)
    P = agg_k(Msp, srcf, dstf, nrm)

    return _tc_final(P, b3.reshape(1, H), btp,
                     Wl1, bl1.reshape(1, Wl1.shape[1]),
                     Wl2, bl2.reshape(1, Wl2.shape[1]), NP, H)
